# baseline (speedup 1.0000x reference)
import jax
import jax.numpy as jnp
from jax.experimental import pallas as pl
from jax.experimental.pallas import tpu as pltpu

EMB_D = 64
NUM_K = 1024
BPB = 2
HW = 1024
ROWS = BPB * HW


def _vq_block(x_ref, e_ref, etb_ref, q_ref, idx_ref, loss_ref):
    i = pl.program_id(0)

    xc = x_ref[...]
    xt = jnp.transpose(xc, (0, 2, 1)).reshape(ROWS, EMB_D)

    e = e_ref[...]
    e2 = e * (-2.0)
    esq = jnp.sum(e * e, axis=0, keepdims=True)

    xsq = jnp.sum(xt * xt, axis=1, keepdims=True)
    ip2 = jnp.dot(xt, e2, preferred_element_type=jnp.float32)
    d = xsq + ip2 + esq

    idx = jnp.argmin(d, axis=1).astype(jnp.int32)

    kiota = jax.lax.broadcasted_iota(jnp.int32, (ROWS, NUM_K), 1)
    onehot = (kiota == idx[:, None]).astype(jnp.bfloat16)
    eb = e.astype(jnp.bfloat16)

    partial = jnp.float32(0.0)
    for b in range(BPB):
        oh_b = onehot[b * HW:(b + 1) * HW, :]
        q_b = jax.lax.dot_general(
            eb, oh_b,
            dimension_numbers=(((1,), (1,)), ((), ())),
            preferred_element_type=jnp.float32,
        )
        xb = xc[b]
        q_ref[b] = xb + (q_b - xb)
        diff = xb - q_b
        partial = partial + jnp.sum(diff * diff)

    idx_ref[...] = idx.reshape(BPB, 1, HW)

    @pl.when(i == 0)
    def _():
        loss_ref[0, 0] = 0.0

    loss_ref[0, 0] += partial


def kernel(x, e_i_ts):
    B, C, H, W = x.shape
    n = B * H * W

    xr = x.reshape(B, C, H * W)
    etb = e_i_ts.T.astype(jnp.bfloat16)

    q_r, idx3, loss_acc = pl.pallas_call(
        _vq_block,
        grid=(B // BPB,),
        in_specs=[
            pl.BlockSpec((BPB, C, HW), lambda i: (i, 0, 0)),
            pl.BlockSpec((C, NUM_K), lambda i: (0, 0)),
            pl.BlockSpec((NUM_K, C), lambda i: (0, 0)),
        ],
        out_specs=[
            pl.BlockSpec((BPB, C, HW), lambda i: (i, 0, 0)),
            pl.BlockSpec((BPB, 1, HW), lambda i: (i, 0, 0)),
            pl.BlockSpec((1, 1), lambda i: (0, 0), memory_space=pltpu.SMEM),
        ],
        out_shape=[
            jax.ShapeDtypeStruct((B, C, H * W), jnp.float32),
            jax.ShapeDtypeStruct((B, 1, HW), jnp.int32),
            jax.ShapeDtypeStruct((1, 1), jnp.float32),
        ],
    )(xr, e_i_ts, etb)

    quantized_x_st = q_r.reshape(B, C, H, W)
    loss = loss_acc[0, 0] / jnp.float32(n * C)
    encoding_indices = idx3.reshape(B, H * W)
    return (quantized_x_st, loss, loss, encoding_indices)


# speedup vs baseline: 1.0032x; 1.0032x over previous
import jax
import jax.numpy as jnp
from jax.experimental import pallas as pl
from jax.experimental.pallas import tpu as pltpu

EMB_D = 64
NUM_K = 1024
BPB = 4
HW = 1024
ROWS = BPB * HW


def _vq_block(x_ref, e_ref, etb_ref, q_ref, idx_ref, loss_ref):
    i = pl.program_id(0)

    xc = x_ref[...]
    xt = jnp.transpose(xc, (0, 2, 1)).reshape(ROWS, EMB_D)

    e = e_ref[...]
    e2 = e * (-2.0)
    esq = jnp.sum(e * e, axis=0, keepdims=True)

    xsq = jnp.sum(xt * xt, axis=1, keepdims=True)
    ip2 = jnp.dot(xt, e2, preferred_element_type=jnp.float32)
    d = xsq + ip2 + esq

    idx = jnp.argmin(d, axis=1).astype(jnp.int32)

    kiota = jax.lax.broadcasted_iota(jnp.int32, (ROWS, NUM_K), 1)
    onehot = (kiota == idx[:, None]).astype(jnp.bfloat16)
    eb = e.astype(jnp.bfloat16)

    partial = jnp.float32(0.0)
    for b in range(BPB):
        oh_b = onehot[b * HW:(b + 1) * HW, :]
        q_b = jax.lax.dot_general(
            eb, oh_b,
            dimension_numbers=(((1,), (1,)), ((), ())),
            preferred_element_type=jnp.float32,
        )
        xb = xc[b]
        q_ref[b] = xb + (q_b - xb)
        diff = xb - q_b
        partial = partial + jnp.sum(diff * diff)

    idx_ref[...] = idx.reshape(BPB, 1, HW)

    @pl.when(i == 0)
    def _():
        loss_ref[0, 0] = 0.0

    loss_ref[0, 0] += partial


def kernel(x, e_i_ts):
    B, C, H, W = x.shape
    n = B * H * W

    xr = x.reshape(B, C, H * W)
    etb = e_i_ts.T.astype(jnp.bfloat16)

    q_r, idx3, loss_acc = pl.pallas_call(
        _vq_block,
        grid=(B // BPB,),
        in_specs=[
            pl.BlockSpec((BPB, C, HW), lambda i: (i, 0, 0)),
            pl.BlockSpec((C, NUM_K), lambda i: (0, 0)),
            pl.BlockSpec((NUM_K, C), lambda i: (0, 0)),
        ],
        out_specs=[
            pl.BlockSpec((BPB, C, HW), lambda i: (i, 0, 0)),
            pl.BlockSpec((BPB, 1, HW), lambda i: (i, 0, 0)),
            pl.BlockSpec((1, 1), lambda i: (0, 0), memory_space=pltpu.SMEM),
        ],
        out_shape=[
            jax.ShapeDtypeStruct((B, C, H * W), jnp.float32),
            jax.ShapeDtypeStruct((B, 1, HW), jnp.int32),
            jax.ShapeDtypeStruct((1, 1), jnp.float32),
        ],
    )(xr, e_i_ts, etb)

    quantized_x_st = q_r.reshape(B, C, H, W)
    loss = loss_acc[0, 0] / jnp.float32(n * C)
    encoding_indices = idx3.reshape(B, H * W)
    return (quantized_x_st, loss, loss, encoding_indices)


# speedup vs baseline: 1.0257x; 1.0224x over previous
import jax
import jax.numpy as jnp
from jax.experimental import pallas as pl
from jax.experimental.pallas import tpu as pltpu

EMB_D = 64
NUM_K = 1024
BPB = 4
HW = 1024
ROWS = BPB * HW


def _vq_block(x_ref, e_ref, q_ref, idx_ref, loss_ref):
    i = pl.program_id(0)

    xc = x_ref[...]
    xt = jnp.transpose(xc, (0, 2, 1)).reshape(ROWS, EMB_D)

    e = e_ref[...]
    e2 = e * (-2.0)
    esq = jnp.sum(e * e, axis=0, keepdims=True)

    xsq = jnp.sum(xt * xt, axis=1, keepdims=True)
    ip2 = jnp.dot(xt, e2, preferred_element_type=jnp.float32)
    d = xsq + ip2 + esq

    idx = jnp.argmin(d, axis=1).astype(jnp.int32)

    kiota = jax.lax.broadcasted_iota(jnp.int32, (ROWS, NUM_K), 1)
    onehot = (kiota == idx[:, None]).astype(jnp.bfloat16)
    eb = e.astype(jnp.bfloat16)

    partial = jnp.float32(0.0)
    for b in range(BPB):
        oh_b = onehot[b * HW:(b + 1) * HW, :]
        q_b = jax.lax.dot_general(
            eb, oh_b,
            dimension_numbers=(((1,), (1,)), ((), ())),
            preferred_element_type=jnp.float32,
        )
        xb = xc[b]
        q_ref[b] = xb + (q_b - xb)
        diff = xb - q_b
        partial = partial + jnp.sum(diff * diff)

    idx_ref[...] = idx.reshape(BPB, 1, HW)

    @pl.when(i == 0)
    def _():
        loss_ref[0, 0] = 0.0

    loss_ref[0, 0] += partial


def kernel(x, e_i_ts):
    B, C, H, W = x.shape
    n = B * H * W

    xr = x.reshape(B, C, H * W)

    q_r, idx3, loss_acc = pl.pallas_call(
        _vq_block,
        grid=(B // BPB,),
        in_specs=[
            pl.BlockSpec((BPB, C, HW), lambda i: (i, 0, 0)),
            pl.BlockSpec((C, NUM_K), lambda i: (0, 0)),
        ],
        out_specs=[
            pl.BlockSpec((BPB, C, HW), lambda i: (i, 0, 0)),
            pl.BlockSpec((BPB, 1, HW), lambda i: (i, 0, 0)),
            pl.BlockSpec((1, 1), lambda i: (0, 0), memory_space=pltpu.SMEM),
        ],
        out_shape=[
            jax.ShapeDtypeStruct((B, C, H * W), jnp.float32),
            jax.ShapeDtypeStruct((B, 1, HW), jnp.int32),
            jax.ShapeDtypeStruct((1, 1), jnp.float32),
        ],
    )(xr, e_i_ts)

    quantized_x_st = q_r.reshape(B, C, H, W)
    loss = loss_acc[0, 0] / jnp.float32(n * C)
    encoding_indices = idx3.reshape(B, H * W)
    return (quantized_x_st, loss, loss, encoding_indices)
